# Initial kernel scaffold; baseline (speedup 1.0000x reference)
import jax
import jax.numpy as jnp
from jax.experimental import pallas as pl


def kernel(x, edge_index, edge_attr, batch, emb, eW, eb, w1a, b1a, w1b, b1b, w2a, b2a, w2b, b2b, cW, cb):
    raise NotImplementedError("write your pallas kernel here")


# speedup vs baseline: 1.3372x; 1.3372x over previous
import functools

import jax
import jax.numpy as jnp
from jax import lax
from jax.experimental import pallas as pl
from jax.experimental.pallas import tpu as pltpu
from jax.experimental.pallas import tpu_sc as plsc

N = 50000
E = 800000
HID = 64
HH = 32
NCLS_ = 4
NGRAPH = 256
NC, NS, LANES = 2, 16, 16

_MESH = plsc.VectorSubcoreMesh(
    core_axis_name="c", subcore_axis_name="s", num_cores=NC, num_subcores=NS)

CH = 128
NCH_E = E // CH
ITERS_E = -(-NCH_E // NS)

GCH = 80
NCH_G = N // GCH
ITERS_G = -(-NCH_G // NS)

PCH = 1000
NCH_P = N // PCH
ITERS_P = -(-NCH_P // NS)

ROWS_T = N // NS
STG = 125
STG_N = ROWS_T // STG


def _zero_rows(ref, nrows):
    z = jnp.zeros((LANES,), jnp.float32)

    @pl.loop(0, nrows)
    def _(i):
        ref[i, pl.ds(0, LANES)] = z
        ref[i, pl.ds(LANES, LANES)] = z


def _emb_body(x_hbm, embA, embB, outA, outB, idx_v, row_v, sem):
    s = lax.axis_index("s")
    c = lax.axis_index("c")

    @pl.loop(0, ITERS_G)
    def _(j):
        k = s + NS * j

        @pl.when(k < NCH_G)
        def _():
            base = k * GCH
            pltpu.sync_copy(x_hbm.at[pl.ds(base, GCH)], idx_v)

            @pl.when(c == 0)
            def _():
                pltpu.async_copy(embA.at[idx_v], row_v, sem).wait()
                pltpu.sync_copy(row_v, outA.at[pl.ds(base, GCH)])

            @pl.when(c == 1)
            def _():
                pltpu.async_copy(embB.at[idx_v], row_v, sem).wait()
                pltpu.sync_copy(row_v, outB.at[pl.ds(base, GCH)])


_emb_call = pl.kernel(
    _emb_body,
    out_type=(jax.ShapeDtypeStruct((N, HH), jnp.float32),) * 2,
    mesh=_MESH,
    compiler_params=pltpu.CompilerParams(use_tc_tiling_on_sc=False),
    scratch_types=[
        pltpu.VMEM((GCH,), jnp.int32),
        pltpu.VMEM((GCH, HH), jnp.float32),
        pltpu.SemaphoreType.DMA,
    ],
)


def _conv_body(src_hbm, dst_hbm, hA, hB, eA, eB, outA, outB,
               sidx, didx, hbuf, ebuf, stage, sem, agg_sh):
    s = lax.axis_index("s")
    c = lax.axis_index("c")

    _zero_rows(stage, STG)
    r0 = s * ROWS_T

    @pl.loop(0, STG_N)
    def _(t):
        pltpu.sync_copy(stage, agg_sh.at[pl.ds(r0 + t * STG, STG)])

    plsc.subcore_barrier()

    @pl.loop(0, ITERS_E)
    def _(j):
        k = s + NS * j

        @pl.when(k < NCH_E)
        def _():
            base = k * CH
            pltpu.sync_copy(src_hbm.at[pl.ds(base, CH)], sidx)
            pltpu.sync_copy(dst_hbm.at[pl.ds(base, CH)], didx)

            @pl.when(c == 0)
            def _():
                pltpu.sync_copy(eA.at[pl.ds(base, CH)], ebuf)
                pltpu.async_copy(hA.at[sidx], hbuf, sem).wait()

            @pl.when(c == 1)
            def _():
                pltpu.sync_copy(eB.at[pl.ds(base, CH)], ebuf)
                pltpu.async_copy(hB.at[sidx], hbuf, sem).wait()

            @plsc.parallel_loop(0, CH, unroll=8)
            def _(i):
                v0 = hbuf[i, pl.ds(0, LANES)] + ebuf[i, pl.ds(0, LANES)]
                hbuf[i, pl.ds(0, LANES)] = jnp.maximum(v0, 0.0)
                v1 = hbuf[i, pl.ds(LANES, LANES)] + ebuf[i, pl.ds(LANES, LANES)]
                hbuf[i, pl.ds(LANES, LANES)] = jnp.maximum(v1, 0.0)

            pltpu.sync_copy(hbuf, agg_sh.at[didx], add=True)

    plsc.subcore_barrier()

    @pl.loop(0, STG_N)
    def _(t):
        off = t * STG
        pltpu.sync_copy(agg_sh.at[pl.ds(r0 + off, STG)], stage)

        @pl.when(c == 0)
        def _():
            pltpu.sync_copy(stage, outA.at[pl.ds(r0 + off, STG)])

        @pl.when(c == 1)
        def _():
            pltpu.sync_copy(stage, outB.at[pl.ds(r0 + off, STG)])


_conv_call = pl.kernel(
    _conv_body,
    out_type=(jax.ShapeDtypeStruct((N, HH), jnp.float32),) * 2,
    mesh=_MESH,
    compiler_params=pltpu.CompilerParams(use_tc_tiling_on_sc=False),
    scratch_types=[
        pltpu.VMEM((CH,), jnp.int32),
        pltpu.VMEM((CH,), jnp.int32),
        pltpu.VMEM((CH, HH), jnp.float32),
        pltpu.VMEM((CH, HH), jnp.float32),
        pltpu.VMEM((STG, HH), jnp.float32),
        pltpu.SemaphoreType.DMA,
        pltpu.VMEM_SHARED((N, HH), jnp.float32),
    ],
)


def _pool_body(batch_hbm, hA, hB, outA, outB,
               bv, rows, pool_l, red, obuf, pool_sh):
    s = lax.axis_index("s")
    c = lax.axis_index("c")

    _zero_rows(pool_l, NGRAPH)

    @pl.loop(0, ITERS_P)
    def _(j):
        k = s + NS * j

        @pl.when(k < NCH_P)
        def _():
            base = k * PCH
            pltpu.sync_copy(batch_hbm.at[pl.ds(base, PCH)], bv.at[pl.ds(0, PCH)])

            @pl.when(c == 0)
            def _():
                pltpu.sync_copy(hA.at[pl.ds(base, PCH)], rows)

            @pl.when(c == 1)
            def _():
                pltpu.sync_copy(hB.at[pl.ds(base, PCH)], rows)

            @pl.loop(0, PCH)
            def _(i):
                g = bv[pl.ds(i, LANES)][0]
                pool_l[g, pl.ds(0, LANES)] = jnp.maximum(
                    pool_l[g, pl.ds(0, LANES)], rows[i, pl.ds(0, LANES)])
                pool_l[g, pl.ds(LANES, LANES)] = jnp.maximum(
                    pool_l[g, pl.ds(LANES, LANES)], rows[i, pl.ds(LANES, LANES)])

    pltpu.sync_copy(pool_l, pool_sh.at[s])
    plsc.subcore_barrier()

    g0 = s * (NGRAPH // NS)
    GG = NGRAPH // NS

    @pl.loop(0, NS)
    def _(t):
        pltpu.sync_copy(pool_sh.at[t, pl.ds(g0, GG)], red.at[t])

    @pl.loop(0, GG)
    def _(g):
        obuf[g, pl.ds(0, LANES)] = red[0, g, pl.ds(0, LANES)]
        obuf[g, pl.ds(LANES, LANES)] = red[0, g, pl.ds(LANES, LANES)]

        @pl.loop(1, NS)
        def _(t):
            obuf[g, pl.ds(0, LANES)] = jnp.maximum(
                obuf[g, pl.ds(0, LANES)], red[t, g, pl.ds(0, LANES)])
            obuf[g, pl.ds(LANES, LANES)] = jnp.maximum(
                obuf[g, pl.ds(LANES, LANES)], red[t, g, pl.ds(LANES, LANES)])

    @pl.when(c == 0)
    def _():
        pltpu.sync_copy(obuf, outA.at[pl.ds(g0, GG)])

    @pl.when(c == 1)
    def _():
        pltpu.sync_copy(obuf, outB.at[pl.ds(g0, GG)])


_pool_call = pl.kernel(
    _pool_body,
    out_type=(jax.ShapeDtypeStruct((NGRAPH, HH), jnp.float32),) * 2,
    mesh=_MESH,
    compiler_params=pltpu.CompilerParams(use_tc_tiling_on_sc=False),
    scratch_types=[
        pltpu.VMEM((PCH + LANES,), jnp.int32),
        pltpu.VMEM((PCH, HH), jnp.float32),
        pltpu.VMEM((NGRAPH, HH), jnp.float32),
        pltpu.VMEM((NS, NGRAPH // NS, HH), jnp.float32),
        pltpu.VMEM((NGRAPH // NS, HH), jnp.float32),
        pltpu.VMEM_SHARED((NS, NGRAPH, HH), jnp.float32),
    ],
)


BE = 4000


def _enc_body(a_ref, w_ref, b_ref, oA_ref, oB_ref):
    e = jnp.dot(a_ref[...], w_ref[...],
                preferred_element_type=jnp.float32) + b_ref[...]
    oA_ref[...] = e[:, :HH]
    oB_ref[...] = e[:, HH:]


_enc_call = pl.pallas_call(
    _enc_body,
    grid=(E // BE,),
    in_specs=[
        pl.BlockSpec((BE, 8), lambda i: (i, 0)),
        pl.BlockSpec((8, HID), lambda i: (0, 0)),
        pl.BlockSpec((1, HID), lambda i: (0, 0)),
    ],
    out_specs=[pl.BlockSpec((BE, HH), lambda i: (i, 0))] * 2,
    out_shape=[jax.ShapeDtypeStruct((E, HH), jnp.float32)] * 2,
)


BN = 2000


def _mlp_body(hA, hB, aA, aB, w1, b1, w2, b2, oA, oB):
    x = (jnp.concatenate([hA[...], hB[...]], axis=1)
         + jnp.concatenate([aA[...], aB[...]], axis=1))
    t = jnp.maximum(
        jnp.dot(x, w1[...], preferred_element_type=jnp.float32) + b1[...], 0.0)
    y = jnp.maximum(
        jnp.dot(t, w2[...], preferred_element_type=jnp.float32) + b2[...], 0.0)
    oA[...] = y[:, :HH]
    oB[...] = y[:, HH:]


_mlp_call = pl.pallas_call(
    _mlp_body,
    grid=(N // BN,),
    in_specs=[
        pl.BlockSpec((BN, HH), lambda i: (i, 0)),
        pl.BlockSpec((BN, HH), lambda i: (i, 0)),
        pl.BlockSpec((BN, HH), lambda i: (i, 0)),
        pl.BlockSpec((BN, HH), lambda i: (i, 0)),
        pl.BlockSpec((HID, HID), lambda i: (0, 0)),
        pl.BlockSpec((1, HID), lambda i: (0, 0)),
        pl.BlockSpec((HID, HID), lambda i: (0, 0)),
        pl.BlockSpec((1, HID), lambda i: (0, 0)),
    ],
    out_specs=[pl.BlockSpec((BN, HH), lambda i: (i, 0))] * 2,
    out_shape=[jax.ShapeDtypeStruct((N, HH), jnp.float32)] * 2,
)


def _cls_body(pA, pB, w_ref, b_ref, o_ref):
    p = jnp.concatenate([pA[...], pB[...]], axis=1)
    o_ref[...] = jnp.dot(p, w_ref[...],
                         preferred_element_type=jnp.float32) + b_ref[...]


_cls_call = pl.pallas_call(
    _cls_body,
    grid=(1,),
    in_specs=[
        pl.BlockSpec((NGRAPH, HH), lambda i: (0, 0)),
        pl.BlockSpec((NGRAPH, HH), lambda i: (0, 0)),
        pl.BlockSpec((HID, 128), lambda i: (0, 0)),
        pl.BlockSpec((1, 128), lambda i: (0, 0)),
    ],
    out_specs=pl.BlockSpec((NGRAPH, 128), lambda i: (0, 0)),
    out_shape=jax.ShapeDtypeStruct((NGRAPH, 128), jnp.float32),
)


@jax.jit
def kernel(x, edge_index, edge_attr, batch, emb, eW, eb,
           w1a, b1a, w1b, b1b, w2a, b2a, w2b, b2b, cW, cb):
    i32 = jnp.int32
    x = x.astype(i32)
    src = edge_index[0].astype(i32)
    dst = edge_index[1].astype(i32)
    batch = batch.astype(i32)

    embA = emb[:, :HH]
    embB = emb[:, HH:]
    a8 = jnp.pad(edge_attr, ((0, 0), (0, 5)))
    eW8 = jnp.pad(eW, ((0, 5), (0, 0)))
    eb2 = eb.reshape(1, HID)

    eA, eB = _enc_call(a8, eW8, eb2)
    h0A, h0B = _emb_call(x, embA, embB)

    agA, agB = _conv_call(src, dst, h0A, h0B, eA, eB)
    h1A, h1B = _mlp_call(h0A, h0B, agA, agB, w1a, b1a.reshape(1, HID),
                         w1b, b1b.reshape(1, HID))

    agA2, agB2 = _conv_call(src, dst, h1A, h1B, eA, eB)
    h2A, h2B = _mlp_call(h1A, h1B, agA2, agB2, w2a, b2a.reshape(1, HID),
                         w2b, b2b.reshape(1, HID))

    pA, pB = _pool_call(batch, h2A, h2B)

    cWp = jnp.pad(cW, ((0, 0), (0, 128 - NCLS_)))
    cbp = jnp.pad(cb, (0, 128 - NCLS_)).reshape(1, 128)
    logits = _cls_call(pA, pB, cWp, cbp)[:, :NCLS_]
    return logits


# speedup vs baseline: 2.3192x; 1.7344x over previous
import functools

import jax
import jax.numpy as jnp
from jax import lax
from jax.experimental import pallas as pl
from jax.experimental.pallas import tpu as pltpu
from jax.experimental.pallas import tpu_sc as plsc

N = 50000
E = 800000
HID = 64
HH = 32
NCLS_ = 4
NGRAPH = 256
NC, NS, LANES = 2, 16, 16

_MESH = plsc.VectorSubcoreMesh(
    core_axis_name="c", subcore_axis_name="s", num_cores=NC, num_subcores=NS)

CH = 128
NCH_E = E // CH
ITERS_E = -(-NCH_E // NS)

GCH = 80
NCH_G = N // GCH
ITERS_G = -(-NCH_G // NS)

PCH = 1000
NCH_P = N // PCH
ITERS_P = -(-NCH_P // NS)

ROWS_T = N // NS
STG = 125
STG_N = ROWS_T // STG


def _zero_rows(ref, nrows):
    z = jnp.zeros((LANES,), jnp.float32)

    @pl.loop(0, nrows)
    def _(i):
        ref[i, pl.ds(0, LANES)] = z
        ref[i, pl.ds(LANES, LANES)] = z


def _emb_body(x_hbm, embA, embB, outA, outB, idx_v, row_v, sem):
    s = lax.axis_index("s")
    c = lax.axis_index("c")

    @pl.loop(0, ITERS_G)
    def _(j):
        k = s + NS * j

        @pl.when(k < NCH_G)
        def _():
            base = k * GCH
            pltpu.sync_copy(x_hbm.at[pl.ds(base, GCH)], idx_v)

            @pl.when(c == 0)
            def _():
                pltpu.async_copy(embA.at[idx_v], row_v, sem).wait()
                pltpu.sync_copy(row_v, outA.at[pl.ds(base, GCH)])

            @pl.when(c == 1)
            def _():
                pltpu.async_copy(embB.at[idx_v], row_v, sem).wait()
                pltpu.sync_copy(row_v, outB.at[pl.ds(base, GCH)])


_emb_call = pl.kernel(
    _emb_body,
    out_type=(jax.ShapeDtypeStruct((N, HH), jnp.float32),) * 2,
    mesh=_MESH,
    compiler_params=pltpu.CompilerParams(use_tc_tiling_on_sc=False),
    scratch_types=[
        pltpu.VMEM((GCH,), jnp.int32),
        pltpu.VMEM((GCH, HH), jnp.float32),
        pltpu.SemaphoreType.DMA,
    ],
)


def _conv_body(src_hbm, dst_hbm, hA, hB, eA, eB, outA, outB,
               sidx, didx, hbuf, ebuf, stage, sem, agg_sh):
    s = lax.axis_index("s")
    c = lax.axis_index("c")

    _zero_rows(stage, STG)
    r0 = s * ROWS_T

    @pl.loop(0, STG_N)
    def _(t):
        pltpu.sync_copy(stage, agg_sh.at[pl.ds(r0 + t * STG, STG)])

    plsc.subcore_barrier()

    @pl.loop(0, ITERS_E)
    def _(j):
        k = s + NS * j

        @pl.when(k < NCH_E)
        def _():
            base = k * CH
            pltpu.sync_copy(src_hbm.at[pl.ds(base, CH)], sidx)
            pltpu.sync_copy(dst_hbm.at[pl.ds(base, CH)], didx)

            @pl.when(c == 0)
            def _():
                pltpu.sync_copy(eA.at[pl.ds(base, CH)], ebuf)
                pltpu.async_copy(hA.at[sidx], hbuf, sem).wait()

            @pl.when(c == 1)
            def _():
                pltpu.sync_copy(eB.at[pl.ds(base, CH)], ebuf)
                pltpu.async_copy(hB.at[sidx], hbuf, sem).wait()

            @plsc.parallel_loop(0, CH, unroll=8)
            def _(i):
                v0 = hbuf[i, pl.ds(0, LANES)] + ebuf[i, pl.ds(0, LANES)]
                hbuf[i, pl.ds(0, LANES)] = jnp.maximum(v0, 0.0)
                v1 = hbuf[i, pl.ds(LANES, LANES)] + ebuf[i, pl.ds(LANES, LANES)]
                hbuf[i, pl.ds(LANES, LANES)] = jnp.maximum(v1, 0.0)

            pltpu.sync_copy(hbuf, agg_sh.at[didx], add=True)

    plsc.subcore_barrier()

    @pl.loop(0, STG_N)
    def _(t):
        off = t * STG
        pltpu.sync_copy(agg_sh.at[pl.ds(r0 + off, STG)], stage)

        @pl.when(c == 0)
        def _():
            pltpu.sync_copy(stage, outA.at[pl.ds(r0 + off, STG)])

        @pl.when(c == 1)
        def _():
            pltpu.sync_copy(stage, outB.at[pl.ds(r0 + off, STG)])


_conv_call = pl.kernel(
    _conv_body,
    out_type=(jax.ShapeDtypeStruct((N, HH), jnp.float32),) * 2,
    mesh=_MESH,
    compiler_params=pltpu.CompilerParams(use_tc_tiling_on_sc=False),
    scratch_types=[
        pltpu.VMEM((CH,), jnp.int32),
        pltpu.VMEM((CH,), jnp.int32),
        pltpu.VMEM((CH, HH), jnp.float32),
        pltpu.VMEM((CH, HH), jnp.float32),
        pltpu.VMEM((STG, HH), jnp.float32),
        pltpu.SemaphoreType.DMA,
        pltpu.VMEM_SHARED((N, HH), jnp.float32),
    ],
)


def _pool_body(batch_hbm, hA, hB, outA, outB,
               bv, rows, pool_l, red, obuf, pool_sh):
    s = lax.axis_index("s")
    c = lax.axis_index("c")

    _zero_rows(pool_l, NGRAPH)

    @pl.loop(0, ITERS_P)
    def _(j):
        k = s + NS * j

        @pl.when(k < NCH_P)
        def _():
            base = k * PCH
            pltpu.sync_copy(batch_hbm.at[pl.ds(base, PCH)], bv.at[pl.ds(0, PCH)])

            @pl.when(c == 0)
            def _():
                pltpu.sync_copy(hA.at[pl.ds(base, PCH)], rows)

            @pl.when(c == 1)
            def _():
                pltpu.sync_copy(hB.at[pl.ds(base, PCH)], rows)

            @pl.loop(0, PCH)
            def _(i):
                g = bv[pl.ds(i, LANES)][0]
                pool_l[g, pl.ds(0, LANES)] = jnp.maximum(
                    pool_l[g, pl.ds(0, LANES)], rows[i, pl.ds(0, LANES)])
                pool_l[g, pl.ds(LANES, LANES)] = jnp.maximum(
                    pool_l[g, pl.ds(LANES, LANES)], rows[i, pl.ds(LANES, LANES)])

    pltpu.sync_copy(pool_l, pool_sh.at[s])
    plsc.subcore_barrier()

    g0 = s * (NGRAPH // NS)
    GG = NGRAPH // NS

    @pl.loop(0, NS)
    def _(t):
        pltpu.sync_copy(pool_sh.at[t, pl.ds(g0, GG)], red.at[t])

    @pl.loop(0, GG)
    def _(g):
        obuf[g, pl.ds(0, LANES)] = red[0, g, pl.ds(0, LANES)]
        obuf[g, pl.ds(LANES, LANES)] = red[0, g, pl.ds(LANES, LANES)]

        @pl.loop(1, NS)
        def _(t):
            obuf[g, pl.ds(0, LANES)] = jnp.maximum(
                obuf[g, pl.ds(0, LANES)], red[t, g, pl.ds(0, LANES)])
            obuf[g, pl.ds(LANES, LANES)] = jnp.maximum(
                obuf[g, pl.ds(LANES, LANES)], red[t, g, pl.ds(LANES, LANES)])

    @pl.when(c == 0)
    def _():
        pltpu.sync_copy(obuf, outA.at[pl.ds(g0, GG)])

    @pl.when(c == 1)
    def _():
        pltpu.sync_copy(obuf, outB.at[pl.ds(g0, GG)])


_pool_call = pl.kernel(
    _pool_body,
    out_type=(jax.ShapeDtypeStruct((NGRAPH, HH), jnp.float32),) * 2,
    mesh=_MESH,
    compiler_params=pltpu.CompilerParams(use_tc_tiling_on_sc=False),
    scratch_types=[
        pltpu.VMEM((PCH + LANES,), jnp.int32),
        pltpu.VMEM((PCH, HH), jnp.float32),
        pltpu.VMEM((NGRAPH, HH), jnp.float32),
        pltpu.VMEM((NS, NGRAPH // NS, HH), jnp.float32),
        pltpu.VMEM((NGRAPH // NS, HH), jnp.float32),
        pltpu.VMEM_SHARED((NS, NGRAPH, HH), jnp.float32),
    ],
)


BE = 6400


def _enc_body(aT_ref, w_ref, b_ref, oA_ref, oB_ref):
    e = lax.dot_general(aT_ref[...], w_ref[...],
                        (((0,), (0,)), ((), ())),
                        preferred_element_type=jnp.float32) + b_ref[...]
    oA_ref[...] = e[:, :HH]
    oB_ref[...] = e[:, HH:]


_enc_call = pl.pallas_call(
    _enc_body,
    grid=(E // BE,),
    in_specs=[
        pl.BlockSpec((3, BE), lambda i: (0, i)),
        pl.BlockSpec((3, HID), lambda i: (0, 0)),
        pl.BlockSpec((1, HID), lambda i: (0, 0)),
    ],
    out_specs=[pl.BlockSpec((BE, HH), lambda i: (i, 0))] * 2,
    out_shape=[jax.ShapeDtypeStruct((E, HH), jnp.float32)] * 2,
)


BN = 2000


def _mlp_body(hA, hB, aA, aB, w1, b1, w2, b2, oA, oB):
    x = (jnp.concatenate([hA[...], hB[...]], axis=1)
         + jnp.concatenate([aA[...], aB[...]], axis=1))
    t = jnp.maximum(
        jnp.dot(x, w1[...], preferred_element_type=jnp.float32) + b1[...], 0.0)
    y = jnp.maximum(
        jnp.dot(t, w2[...], preferred_element_type=jnp.float32) + b2[...], 0.0)
    oA[...] = y[:, :HH]
    oB[...] = y[:, HH:]


_mlp_call = pl.pallas_call(
    _mlp_body,
    grid=(N // BN,),
    in_specs=[
        pl.BlockSpec((BN, HH), lambda i: (i, 0)),
        pl.BlockSpec((BN, HH), lambda i: (i, 0)),
        pl.BlockSpec((BN, HH), lambda i: (i, 0)),
        pl.BlockSpec((BN, HH), lambda i: (i, 0)),
        pl.BlockSpec((HID, HID), lambda i: (0, 0)),
        pl.BlockSpec((1, HID), lambda i: (0, 0)),
        pl.BlockSpec((HID, HID), lambda i: (0, 0)),
        pl.BlockSpec((1, HID), lambda i: (0, 0)),
    ],
    out_specs=[pl.BlockSpec((BN, HH), lambda i: (i, 0))] * 2,
    out_shape=[jax.ShapeDtypeStruct((N, HH), jnp.float32)] * 2,
)


def _cls_body(pA, pB, w_ref, b_ref, o_ref):
    p = jnp.concatenate([pA[...], pB[...]], axis=1)
    o_ref[...] = jnp.dot(p, w_ref[...],
                         preferred_element_type=jnp.float32) + b_ref[...]


_cls_call = pl.pallas_call(
    _cls_body,
    grid=(1,),
    in_specs=[
        pl.BlockSpec((NGRAPH, HH), lambda i: (0, 0)),
        pl.BlockSpec((NGRAPH, HH), lambda i: (0, 0)),
        pl.BlockSpec((HID, 128), lambda i: (0, 0)),
        pl.BlockSpec((1, 128), lambda i: (0, 0)),
    ],
    out_specs=pl.BlockSpec((NGRAPH, 128), lambda i: (0, 0)),
    out_shape=jax.ShapeDtypeStruct((NGRAPH, 128), jnp.float32),
)


@jax.jit
def kernel(x, edge_index, edge_attr, batch, emb, eW, eb,
           w1a, b1a, w1b, b1b, w2a, b2a, w2b, b2b, cW, cb):
    i32 = jnp.int32
    x = x.astype(i32)
    src = edge_index[0].astype(i32)
    dst = edge_index[1].astype(i32)
    batch = batch.astype(i32)

    embA = emb[:, :HH]
    embB = emb[:, HH:]
    eb2 = eb.reshape(1, HID)

    eA, eB = _enc_call(edge_attr.T, eW, eb2)
    h0A, h0B = _emb_call(x, embA, embB)

    agA, agB = _conv_call(src, dst, h0A, h0B, eA, eB)
    h1A, h1B = _mlp_call(h0A, h0B, agA, agB, w1a, b1a.reshape(1, HID),
                         w1b, b1b.reshape(1, HID))

    agA2, agB2 = _conv_call(src, dst, h1A, h1B, eA, eB)
    h2A, h2B = _mlp_call(h1A, h1B, agA2, agB2, w2a, b2a.reshape(1, HID),
                         w2b, b2b.reshape(1, HID))

    pA, pB = _pool_call(batch, h2A, h2B)

    cWp = jnp.pad(cW, ((0, 0), (0, 128 - NCLS_)))
    cbp = jnp.pad(cb, (0, 128 - NCLS_)).reshape(1, 128)
    logits = _cls_call(pA, pB, cWp, cbp)[:, :NCLS_]
    return logits


# speedup vs baseline: 4.1939x; 1.8083x over previous
import functools

import jax
import jax.numpy as jnp
from jax import lax
from jax.experimental import pallas as pl
from jax.experimental.pallas import tpu as pltpu
from jax.experimental.pallas import tpu_sc as plsc

N = 50000
E = 800000
HID = 64
HH = 32
NCLS_ = 4
NGRAPH = 256
NC, NS, LANES = 2, 16, 16

_MESH = plsc.VectorSubcoreMesh(
    core_axis_name="c", subcore_axis_name="s", num_cores=NC, num_subcores=NS)

CH = 128
NCH_E = E // CH
SUP = 10
NSUP = NCH_E // SUP
ITERS_S = -(-NSUP // NS)

GCH = 80
NCH_G = N // GCH
ITERS_G = -(-NCH_G // NS)

PCH = 1000
NCH_P = N // PCH
ITERS_P = -(-NCH_P // NS)

ROWS_T = N // NS
STG = 125
STG_N = ROWS_T // STG


def _zero_rows(ref, nrows):
    z = jnp.zeros((LANES,), jnp.float32)

    @pl.loop(0, nrows)
    def _(i):
        ref[i, pl.ds(0, LANES)] = z
        ref[i, pl.ds(LANES, LANES)] = z


def _emb_body(x_hbm, embA, embB, outA, outB, idx_v, row_v, sem):
    s = lax.axis_index("s")
    c = lax.axis_index("c")

    @pl.loop(0, ITERS_G)
    def _(j):
        k = s + NS * j

        @pl.when(k < NCH_G)
        def _():
            base = k * GCH
            pltpu.sync_copy(x_hbm.at[pl.ds(base, GCH)], idx_v)

            @pl.when(c == 0)
            def _():
                pltpu.async_copy(embA.at[idx_v], row_v, sem).wait()
                pltpu.sync_copy(row_v, outA.at[pl.ds(base, GCH)])

            @pl.when(c == 1)
            def _():
                pltpu.async_copy(embB.at[idx_v], row_v, sem).wait()
                pltpu.sync_copy(row_v, outB.at[pl.ds(base, GCH)])


_emb_call = pl.kernel(
    _emb_body,
    out_type=(jax.ShapeDtypeStruct((N, HH), jnp.float32),) * 2,
    mesh=_MESH,
    compiler_params=pltpu.CompilerParams(use_tc_tiling_on_sc=False),
    scratch_types=[
        pltpu.VMEM((GCH,), jnp.int32),
        pltpu.VMEM((GCH, HH), jnp.float32),
        pltpu.SemaphoreType.DMA,
    ],
)


def _conv_body(src2_hbm, dst2_hbm, hA, hB, eA, eB, outA, outB,
               sidxb, didxb, h0b, h1b, e0b, e1b, stage,
               sg0, sg1, se0, se1, agg_sh):
    s = lax.axis_index("s")
    c = lax.axis_index("c")

    _zero_rows(stage, STG)
    r0 = s * ROWS_T

    @pl.loop(0, STG_N)
    def _(t):
        pltpu.sync_copy(stage, agg_sh.at[pl.ds(r0 + t * STG, STG)])

    plsc.subcore_barrier()

    hb = (h0b, h1b)
    eb_ = (e0b, e1b)
    sg = (sg0, sg1)
    se = (se0, se1)

    @pl.loop(0, ITERS_S)
    def _(m):
        k = s + NS * m

        @pl.when(k < NSUP)
        def _():
            cbase = k * SUP
            ebase = cbase * CH
            pltpu.sync_copy(src2_hbm.at[pl.ds(cbase, SUP)], sidxb)
            pltpu.sync_copy(dst2_hbm.at[pl.ds(cbase, SUP)], didxb)

            def _issue(b):
                p = b & 1
                ebs = pl.ds(ebase + b * CH, CH)

                @pl.when(c == 0)
                def _():
                    pltpu.async_copy(hA.at[sidxb.at[b]], hb[p], sg[p])
                    pltpu.async_copy(eA.at[ebs], eb_[p], se[p])

                @pl.when(c == 1)
                def _():
                    pltpu.async_copy(hB.at[sidxb.at[b]], hb[p], sg[p])
                    pltpu.async_copy(eB.at[ebs], eb_[p], se[p])

            def _wait(b):
                p = b & 1
                pltpu.make_async_copy(hA.at[sidxb.at[b]], hb[p], sg[p]).wait()
                pltpu.make_async_copy(
                    eA.at[pl.ds(ebase + b * CH, CH)], eb_[p], se[p]).wait()

            _issue(0)
            for b in range(SUP):
                p = b & 1
                if b + 1 < SUP:
                    _issue(b + 1)
                _wait(b)
                hbp = hb[p]
                ebp = eb_[p]

                @plsc.parallel_loop(0, CH, unroll=8)
                def _(i):
                    v0 = hbp[i, pl.ds(0, LANES)] + ebp[i, pl.ds(0, LANES)]
                    hbp[i, pl.ds(0, LANES)] = jnp.maximum(v0, 0.0)
                    v1 = hbp[i, pl.ds(LANES, LANES)] + ebp[i, pl.ds(LANES, LANES)]
                    hbp[i, pl.ds(LANES, LANES)] = jnp.maximum(v1, 0.0)

                pltpu.sync_copy(hbp, agg_sh.at[didxb.at[b]], add=True)

    plsc.subcore_barrier()

    @pl.loop(0, STG_N)
    def _(t):
        off = t * STG
        pltpu.sync_copy(agg_sh.at[pl.ds(r0 + off, STG)], stage)

        @pl.when(c == 0)
        def _():
            pltpu.sync_copy(stage, outA.at[pl.ds(r0 + off, STG)])

        @pl.when(c == 1)
        def _():
            pltpu.sync_copy(stage, outB.at[pl.ds(r0 + off, STG)])


_conv_call = pl.kernel(
    _conv_body,
    out_type=(jax.ShapeDtypeStruct((N, HH), jnp.float32),) * 2,
    mesh=_MESH,
    compiler_params=pltpu.CompilerParams(use_tc_tiling_on_sc=False),
    scratch_types=[
        pltpu.VMEM((SUP, CH), jnp.int32),
        pltpu.VMEM((SUP, CH), jnp.int32),
        pltpu.VMEM((CH, HH), jnp.float32),
        pltpu.VMEM((CH, HH), jnp.float32),
        pltpu.VMEM((CH, HH), jnp.float32),
        pltpu.VMEM((CH, HH), jnp.float32),
        pltpu.VMEM((STG, HH), jnp.float32),
        pltpu.SemaphoreType.DMA,
        pltpu.SemaphoreType.DMA,
        pltpu.SemaphoreType.DMA,
        pltpu.SemaphoreType.DMA,
        pltpu.VMEM_SHARED((N, HH), jnp.float32),
    ],
)


def _pool_body(batch_hbm, hA, hB, outA, outB,
               bv, rows, pool_l, red, obuf, pool_sh):
    s = lax.axis_index("s")
    c = lax.axis_index("c")

    _zero_rows(pool_l, NGRAPH)

    @pl.loop(0, ITERS_P)
    def _(j):
        k = s + NS * j

        @pl.when(k < NCH_P)
        def _():
            base = k * PCH
            pltpu.sync_copy(batch_hbm.at[pl.ds(base, PCH)], bv.at[pl.ds(0, PCH)])

            @pl.when(c == 0)
            def _():
                pltpu.sync_copy(hA.at[pl.ds(base, PCH)], rows)

            @pl.when(c == 1)
            def _():
                pltpu.sync_copy(hB.at[pl.ds(base, PCH)], rows)

            @pl.loop(0, PCH)
            def _(i):
                g = bv[pl.ds(i, LANES)][0]
                pool_l[g, pl.ds(0, LANES)] = jnp.maximum(
                    pool_l[g, pl.ds(0, LANES)], rows[i, pl.ds(0, LANES)])
                pool_l[g, pl.ds(LANES, LANES)] = jnp.maximum(
                    pool_l[g, pl.ds(LANES, LANES)], rows[i, pl.ds(LANES, LANES)])

    pltpu.sync_copy(pool_l, pool_sh.at[s])
    plsc.subcore_barrier()

    g0 = s * (NGRAPH // NS)
    GG = NGRAPH // NS

    @pl.loop(0, NS)
    def _(t):
        pltpu.sync_copy(pool_sh.at[t, pl.ds(g0, GG)], red.at[t])

    @pl.loop(0, GG)
    def _(g):
        obuf[g, pl.ds(0, LANES)] = red[0, g, pl.ds(0, LANES)]
        obuf[g, pl.ds(LANES, LANES)] = red[0, g, pl.ds(LANES, LANES)]

        @pl.loop(1, NS)
        def _(t):
            obuf[g, pl.ds(0, LANES)] = jnp.maximum(
                obuf[g, pl.ds(0, LANES)], red[t, g, pl.ds(0, LANES)])
            obuf[g, pl.ds(LANES, LANES)] = jnp.maximum(
                obuf[g, pl.ds(LANES, LANES)], red[t, g, pl.ds(LANES, LANES)])

    @pl.when(c == 0)
    def _():
        pltpu.sync_copy(obuf, outA.at[pl.ds(g0, GG)])

    @pl.when(c == 1)
    def _():
        pltpu.sync_copy(obuf, outB.at[pl.ds(g0, GG)])


_pool_call = pl.kernel(
    _pool_body,
    out_type=(jax.ShapeDtypeStruct((NGRAPH, HH), jnp.float32),) * 2,
    mesh=_MESH,
    compiler_params=pltpu.CompilerParams(use_tc_tiling_on_sc=False),
    scratch_types=[
        pltpu.VMEM((PCH + LANES,), jnp.int32),
        pltpu.VMEM((PCH, HH), jnp.float32),
        pltpu.VMEM((NGRAPH, HH), jnp.float32),
        pltpu.VMEM((NS, NGRAPH // NS, HH), jnp.float32),
        pltpu.VMEM((NGRAPH // NS, HH), jnp.float32),
        pltpu.VMEM_SHARED((NS, NGRAPH, HH), jnp.float32),
    ],
)


BE = 6400


def _enc_body(aT_ref, w_ref, b_ref, oA_ref, oB_ref):
    e = lax.dot_general(aT_ref[...], w_ref[...],
                        (((0,), (0,)), ((), ())),
                        preferred_element_type=jnp.float32) + b_ref[...]
    oA_ref[...] = e[:, :HH]
    oB_ref[...] = e[:, HH:]


_enc_call = pl.pallas_call(
    _enc_body,
    grid=(E // BE,),
    in_specs=[
        pl.BlockSpec((3, BE), lambda i: (0, i)),
        pl.BlockSpec((3, HID), lambda i: (0, 0)),
        pl.BlockSpec((1, HID), lambda i: (0, 0)),
    ],
    out_specs=[pl.BlockSpec((BE, HH), lambda i: (i, 0))] * 2,
    out_shape=[jax.ShapeDtypeStruct((E, HH), jnp.float32)] * 2,
)


BN = 2000


def _mlp_body(hA, hB, aA, aB, w1, b1, w2, b2, oA, oB):
    x = (jnp.concatenate([hA[...], hB[...]], axis=1)
         + jnp.concatenate([aA[...], aB[...]], axis=1))
    t = jnp.maximum(
        jnp.dot(x, w1[...], preferred_element_type=jnp.float32) + b1[...], 0.0)
    y = jnp.maximum(
        jnp.dot(t, w2[...], preferred_element_type=jnp.float32) + b2[...], 0.0)
    oA[...] = y[:, :HH]
    oB[...] = y[:, HH:]


_mlp_call = pl.pallas_call(
    _mlp_body,
    grid=(N // BN,),
    in_specs=[
        pl.BlockSpec((BN, HH), lambda i: (i, 0)),
        pl.BlockSpec((BN, HH), lambda i: (i, 0)),
        pl.BlockSpec((BN, HH), lambda i: (i, 0)),
        pl.BlockSpec((BN, HH), lambda i: (i, 0)),
        pl.BlockSpec((HID, HID), lambda i: (0, 0)),
        pl.BlockSpec((1, HID), lambda i: (0, 0)),
        pl.BlockSpec((HID, HID), lambda i: (0, 0)),
        pl.BlockSpec((1, HID), lambda i: (0, 0)),
    ],
    out_specs=[pl.BlockSpec((BN, HH), lambda i: (i, 0))] * 2,
    out_shape=[jax.ShapeDtypeStruct((N, HH), jnp.float32)] * 2,
)


def _cls_body(pA, pB, w_ref, b_ref, o_ref):
    p = jnp.concatenate([pA[...], pB[...]], axis=1)
    o_ref[...] = jnp.dot(p, w_ref[...],
                         preferred_element_type=jnp.float32) + b_ref[...]


_cls_call = pl.pallas_call(
    _cls_body,
    grid=(1,),
    in_specs=[
        pl.BlockSpec((NGRAPH, HH), lambda i: (0, 0)),
        pl.BlockSpec((NGRAPH, HH), lambda i: (0, 0)),
        pl.BlockSpec((HID, 128), lambda i: (0, 0)),
        pl.BlockSpec((1, 128), lambda i: (0, 0)),
    ],
    out_specs=pl.BlockSpec((NGRAPH, 128), lambda i: (0, 0)),
    out_shape=jax.ShapeDtypeStruct((NGRAPH, 128), jnp.float32),
)


@jax.jit
def kernel(x, edge_index, edge_attr, batch, emb, eW, eb,
           w1a, b1a, w1b, b1b, w2a, b2a, w2b, b2b, cW, cb):
    i32 = jnp.int32
    x = x.astype(i32)
    src = edge_index[0].astype(i32)
    dst = edge_index[1].astype(i32)
    batch = batch.astype(i32)

    embA = emb[:, :HH]
    embB = emb[:, HH:]
    eb2 = eb.reshape(1, HID)

    eA, eB = _enc_call(edge_attr.T, eW, eb2)
    h0A, h0B = _emb_call(x, embA, embB)

    src2 = src.reshape(NCH_E, CH)
    dst2 = dst.reshape(NCH_E, CH)
    agA, agB = _conv_call(src2, dst2, h0A, h0B, eA, eB)
    h1A, h1B = _mlp_call(h0A, h0B, agA, agB, w1a, b1a.reshape(1, HID),
                         w1b, b1b.reshape(1, HID))

    agA2, agB2 = _conv_call(src2, dst2, h1A, h1B, eA, eB)
    h2A, h2B = _mlp_call(h1A, h1B, agA2, agB2, w2a, b2a.reshape(1, HID),
                         w2b, b2b.reshape(1, HID))

    pA, pB = _pool_call(batch, h2A, h2B)

    cWp = jnp.pad(cW, ((0, 0), (0, 128 - NCLS_)))
    cbp = jnp.pad(cb, (0, 128 - NCLS_)).reshape(1, 128)
    logits = _cls_call(pA, pB, cWp, cbp)[:, :NCLS_]
    return logits
